# baseline (speedup 1.0000x reference)
import functools

import jax
import jax.numpy as jnp
from jax import lax
from jax.experimental import pallas as pl
from jax.experimental.pallas import tpu as pltpu
from jax.experimental.pallas import tpu_sc as plsc

_EMBED_DIM = 128
_CH = 64
_TABLE = 512
_A = 32
_B = 16
_SC_ROWS = 16384
_TC_BLK = 2048


def _tables_body(freq_ref, full_ref, ta_ref, tb_ref):
    freq = freq_ref[0:1, :]
    lane128 = lax.broadcasted_iota(jnp.int32, (1, _EMBED_DIM), 1)
    odd = (lane128 % 2) == 1
    hi = lane128 >= _CH

    pos = lax.broadcasted_iota(jnp.int32, (_TABLE, _CH), 0).astype(jnp.float32)
    arg = pos * freq[:, :_CH]
    full_ref[...] = jnp.where(jnp.logical_not(odd[:, :_CH]),
                              jnp.sin(arg), jnp.cos(arg))

    pos_a = 16.0 * lax.broadcasted_iota(jnp.int32, (_A, _EMBED_DIM), 0).astype(jnp.float32)
    arg_a = pos_a * freq
    use_sin_a = jnp.logical_xor(odd, hi)
    ta_ref[...] = jnp.where(jnp.logical_not(use_sin_a), jnp.sin(arg_a), jnp.cos(arg_a))

    pos_b = lax.broadcasted_iota(jnp.int32, (_B, _EMBED_DIM), 0).astype(jnp.float32)
    arg_b = pos_b * freq
    sgn = jnp.where(odd, -1.0, 1.0).astype(jnp.float32)
    tb_ref[...] = jnp.where(hi, jnp.sin(arg_b) * sgn, jnp.cos(arg_b))


def _build_tables(freq_blk):
    return pl.pallas_call(
        _tables_body,
        out_shape=(
            jax.ShapeDtypeStruct((_TABLE, _CH), jnp.float32),
            jax.ShapeDtypeStruct((_A, _EMBED_DIM), jnp.float32),
            jax.ShapeDtypeStruct((_B, _EMBED_DIM), jnp.float32),
        ),
    )(freq_blk)


def _tc_gather_body(idx_ref, ta_ref, tb_ref, out_ref):
    idx = idx_ref[...]
    a = idx >> 4
    b = idx & 15
    ia = lax.broadcasted_iota(jnp.int32, (_TC_BLK, _A), 1)
    ib = lax.broadcasted_iota(jnp.int32, (_TC_BLK, _B), 1)
    oh_a = (a == ia).astype(jnp.float32)
    oh_b = (b == ib).astype(jnp.float32)
    ga = jnp.dot(oh_a, ta_ref[...], preferred_element_type=jnp.float32)
    gb = jnp.dot(oh_b, tb_ref[...], preferred_element_type=jnp.float32)
    out_ref[...] = (ga[:, :_CH] * gb[:, :_CH]
                    + ga[:, _CH:] * gb[:, _CH:])


def _tc_gather(idx_col, ta, tb):
    n = idx_col.shape[0]
    grid = n // _TC_BLK
    return pl.pallas_call(
        _tc_gather_body,
        grid=(grid,),
        in_specs=[
            pl.BlockSpec((_TC_BLK, 1), lambda i: (i, 0)),
            pl.BlockSpec((_A, _EMBED_DIM), lambda i: (0, 0)),
            pl.BlockSpec((_B, _EMBED_DIM), lambda i: (0, 0)),
        ],
        out_specs=pl.BlockSpec((_TC_BLK, _CH), lambda i: (i, 0)),
        out_shape=jax.ShapeDtypeStruct((n, _CH), jnp.float32),
    )(idx_col, ta, tb)


@functools.cache
def _sc_gather_call(n_idx):
    info = plsc.get_sparse_core_info()
    nc = info.num_cores
    nw = nc * info.num_subcores
    per_w = n_idx // nw
    mesh = plsc.VectorSubcoreMesh(core_axis_name="c", subcore_axis_name="s")

    @functools.partial(
        pl.kernel,
        mesh=mesh,
        out_type=jax.ShapeDtypeStruct((n_idx, _CH), jnp.float32),
        scratch_types=[
            pltpu.VMEM((per_w,), jnp.int32),
            pltpu.VMEM((per_w, _CH), jnp.float32),
            pltpu.SemaphoreType.DMA,
        ],
        compiler_params=pltpu.CompilerParams(use_tc_tiling_on_sc=False),
    )
    def gather(table_hbm, idx_hbm, out_hbm, idx_v, rows_v, sem):
        wid = lax.axis_index("s") * nc + lax.axis_index("c")
        base = wid * per_w
        pltpu.sync_copy(idx_hbm.at[pl.ds(base, per_w)], idx_v)
        pltpu.async_copy(table_hbm.at[idx_v], rows_v, sem).wait()
        pltpu.sync_copy(rows_v, out_hbm.at[pl.ds(base, per_w)])

    return gather


def kernel(coord_idx, inv_freq):
    freq_blk = jnp.broadcast_to(
        jnp.tile(jnp.repeat(inv_freq, 2), 2)[None, :], (8, _EMBED_DIM))
    table, ta, tb = _build_tables(freq_blk)
    n_idx = coord_idx.size
    flat = coord_idx.reshape(n_idx)
    sc_part = _sc_gather_call(_SC_ROWS)(table, flat[:_SC_ROWS])
    tc_part = _tc_gather(flat[_SC_ROWS:].reshape(-1, 1), ta, tb)
    out_flat = jnp.concatenate([sc_part, tc_part], axis=0)
    return out_flat.reshape(n_idx // 2, _EMBED_DIM)


# speedup vs baseline: 2.4879x; 2.4879x over previous
import functools

import jax
import jax.numpy as jnp
from jax import lax
from jax.experimental import pallas as pl
from jax.experimental.pallas import tpu as pltpu
from jax.experimental.pallas import tpu_sc as plsc

_EMBED_DIM = 128
_CH = 64
_TABLE = 512
_A = 32
_B = 16
_SC_ROWS = 16384
_TC_BLK = 2048


def _tables_body(freq_ref, full_ref, ta_ref, tb_ref):
    freq = freq_ref[0:1, :]
    lane128 = lax.broadcasted_iota(jnp.int32, (1, _EMBED_DIM), 1)
    odd = (lane128 % 2) == 1
    hi = lane128 >= _CH

    pos = lax.broadcasted_iota(jnp.int32, (_TABLE, _CH), 0).astype(jnp.float32)
    arg = pos * freq[:, :_CH]
    full_ref[...] = jnp.where(jnp.logical_not(odd[:, :_CH]),
                              jnp.sin(arg), jnp.cos(arg))

    pos_a = 16.0 * lax.broadcasted_iota(jnp.int32, (_A, _EMBED_DIM), 0).astype(jnp.float32)
    arg_a = pos_a * freq
    use_sin_a = jnp.logical_xor(odd, hi)
    ta_ref[...] = jnp.where(jnp.logical_not(use_sin_a), jnp.sin(arg_a), jnp.cos(arg_a))

    pos_b = lax.broadcasted_iota(jnp.int32, (_B, _EMBED_DIM), 0).astype(jnp.float32)
    arg_b = pos_b * freq
    sgn = jnp.where(odd, -1.0, 1.0).astype(jnp.float32)
    tb_ref[...] = jnp.where(hi, jnp.sin(arg_b) * sgn, jnp.cos(arg_b))


def _build_tables(freq_blk):
    return pl.pallas_call(
        _tables_body,
        out_shape=(
            jax.ShapeDtypeStruct((_TABLE, _CH), jnp.float32),
            jax.ShapeDtypeStruct((_A, _EMBED_DIM), jnp.float32),
            jax.ShapeDtypeStruct((_B, _EMBED_DIM), jnp.float32),
        ),
    )(freq_blk)


def _tc_gather_body(idx_ref, ta_ref, tb_ref, out_ref):
    idx = idx_ref[...]
    a = idx >> 4
    b = idx & 15
    ia = lax.broadcasted_iota(jnp.int32, (_TC_BLK, _A), 1)
    ib = lax.broadcasted_iota(jnp.int32, (_TC_BLK, _B), 1)
    oh_a = (a == ia).astype(jnp.float32)
    oh_b = (b == ib).astype(jnp.float32)
    ga = jnp.dot(oh_a, ta_ref[...], preferred_element_type=jnp.float32)
    gb = jnp.dot(oh_b, tb_ref[...], preferred_element_type=jnp.float32)
    out_ref[...] = (ga[:, :_CH] * gb[:, :_CH]
                    + ga[:, _CH:] * gb[:, _CH:])


def _tc_gather(idx_col, ta, tb):
    n = idx_col.shape[0]
    grid = n // _TC_BLK
    return pl.pallas_call(
        _tc_gather_body,
        grid=(grid,),
        in_specs=[
            pl.BlockSpec((_TC_BLK, 1), lambda i: (i, 0)),
            pl.BlockSpec((_A, _EMBED_DIM), lambda i: (0, 0)),
            pl.BlockSpec((_B, _EMBED_DIM), lambda i: (0, 0)),
        ],
        out_specs=pl.BlockSpec((_TC_BLK, _CH), lambda i: (i, 0)),
        out_shape=jax.ShapeDtypeStruct((n, _CH), jnp.float32),
    )(idx_col, ta, tb)


@functools.cache
def _sc_gather_call(n_idx):
    info = plsc.get_sparse_core_info()
    nc = info.num_cores
    nw = nc * info.num_subcores
    per_w = n_idx // nw
    mesh = plsc.VectorSubcoreMesh(core_axis_name="c", subcore_axis_name="s")

    n_out = n_idx // 2
    per_o = n_out // nw

    @functools.partial(
        pl.kernel,
        mesh=mesh,
        out_type=jax.ShapeDtypeStruct((n_out, _EMBED_DIM), jnp.float32),
        scratch_types=[
            pltpu.VMEM((per_o,), jnp.int32),
            pltpu.VMEM((per_o,), jnp.int32),
            pltpu.VMEM((per_o, _CH), jnp.float32),
            pltpu.VMEM((per_o, _CH), jnp.float32),
            pltpu.SemaphoreType.DMA,
            pltpu.SemaphoreType.DMA,
        ],
        compiler_params=pltpu.CompilerParams(use_tc_tiling_on_sc=False),
    )
    def gather(table_hbm, h_hbm, w_hbm, out_hbm, idxh_v, idxw_v, hbuf, wbuf,
               sem_h, sem_w):
        wid = lax.axis_index("s") * nc + lax.axis_index("c")
        base = wid * per_o
        pltpu.sync_copy(h_hbm.at[pl.ds(base, per_o)], idxh_v)
        pltpu.sync_copy(w_hbm.at[pl.ds(base, per_o)], idxw_v)
        ch = pltpu.async_copy(table_hbm.at[idxh_v], hbuf, sem_h)
        cw = pltpu.async_copy(table_hbm.at[idxw_v], wbuf, sem_w)
        ch.wait()
        pltpu.sync_copy(hbuf, out_hbm.at[pl.ds(base, per_o), pl.ds(0, _CH)])
        cw.wait()
        pltpu.sync_copy(wbuf, out_hbm.at[pl.ds(base, per_o), pl.ds(_CH, _CH)])

    return gather


def kernel(coord_idx, inv_freq):
    freq_blk = jnp.broadcast_to(
        jnp.tile(jnp.repeat(inv_freq, 2), 2)[None, :], (8, _EMBED_DIM))
    table, ta, tb = _build_tables(freq_blk)
    n_idx = coord_idx.size
    flat2 = coord_idx.reshape(n_idx // 2, 2)
    return _sc_gather_call(n_idx)(table, flat2[:, 0], flat2[:, 1])
